# Initial kernel scaffold; baseline (speedup 1.0000x reference)
import jax
import jax.numpy as jnp
from jax.experimental import pallas as pl


def kernel(x, pos, batch, x_skip, pos_skip, batch_skip, W, b):
    raise NotImplementedError("write your pallas kernel here")


# speedup vs baseline: 12.2651x; 12.2651x over previous
import jax
import jax.numpy as jnp
from jax import lax
from jax.experimental import pallas as pl


def _proj_kernel(x_ref, w_ref, o_ref):
    o_ref[...] = jnp.dot(x_ref[...], w_ref[...],
                         preferred_element_type=jnp.float32)


def _main_kernel(q_ref, pt_ref, xp_ref, xs_ref, w2_ref, b_ref, o_ref):
    a = q_ref[...]
    q0 = a[:, 0:1]
    q1 = a[:, 1:2]
    q2 = a[:, 2:3]
    bq = a[:, 3:4]
    pt = pt_ref[...]
    p0 = pt[0:1, :]
    p1 = pt[1:2, :]
    p2 = pt[2:3, :]
    bp = pt[3:4, :]
    qn = q0 * q0 + q1 * q1 + q2 * q2
    pn = p0 * p0 + p1 * p1 + p2 * p2
    def _t(v):
        return v.astype(jnp.bfloat16).astype(jnp.float32)
    cross = _t(q0) * _t(p0) + _t(q1) * _t(p1) + _t(q2) * _t(p2)
    d2 = jnp.maximum((qn + pn) - 2.0 * cross, 0.0)
    d2 = jnp.where(bq != bp, 1e10, d2)

    n = d2.shape[1]
    colid = lax.broadcasted_iota(jnp.int32, d2.shape, 1)
    wacc = jnp.zeros_like(d2)
    den = jnp.zeros_like(qn)
    for _ in range(3):
        m = jnp.min(d2, axis=1, keepdims=True)
        cand = jnp.where(d2 == m, colid, n)
        j = jnp.min(cand, axis=1, keepdims=True)
        sel = colid == j
        wk = 1.0 / jnp.maximum(m, 1e-16)
        den = den + wk
        wacc = jnp.where(sel, wk, wacc)
        d2 = jnp.where(sel, 3e10, d2)
    wacc = wacc / den

    out = jnp.dot(wacc, xp_ref[...], preferred_element_type=jnp.float32)
    out += jnp.dot(xs_ref[...], w2_ref[...], preferred_element_type=jnp.float32)
    out += b_ref[0:1, :]
    o_ref[...] = out


def kernel(x, pos, batch, x_skip, pos_skip, batch_skip, W, b):
    M = pos_skip.shape[0]
    N = pos.shape[0]
    F = x.shape[1]
    Fs = x_skip.shape[1]
    BLK = 256

    W1 = W[:F, :]
    W2 = W[F:, :]

    xp = pl.pallas_call(
        _proj_kernel,
        grid=(8,),
        in_specs=[pl.BlockSpec((N // 8, F), lambda i: (i, 0)),
                  pl.BlockSpec((F, F), lambda i: (0, 0))],
        out_specs=pl.BlockSpec((N // 8, F), lambda i: (i, 0)),
        out_shape=jax.ShapeDtypeStruct((N, F), jnp.float32),
    )(x, W1)

    q = jnp.concatenate(
        [pos_skip, batch_skip.astype(jnp.float32)[:, None],
         jnp.zeros((M, 4), jnp.float32)], axis=1)
    pt_arr = jnp.concatenate(
        [pos.T, batch.astype(jnp.float32)[None, :],
         jnp.zeros((4, N), jnp.float32)], axis=0)
    b_arr = jnp.zeros((8, F), jnp.float32).at[0].set(b)

    out = pl.pallas_call(
        _main_kernel,
        grid=(M // BLK,),
        in_specs=[
            pl.BlockSpec((BLK, 8), lambda i: (i, 0)),
            pl.BlockSpec((8, N), lambda i: (0, 0)),
            pl.BlockSpec((N, F), lambda i: (0, 0)),
            pl.BlockSpec((BLK, Fs), lambda i: (i, 0)),
            pl.BlockSpec((Fs, F), lambda i: (0, 0)),
            pl.BlockSpec((8, F), lambda i: (0, 0)),
        ],
        out_specs=pl.BlockSpec((BLK, F), lambda i: (i, 0)),
        out_shape=jax.ShapeDtypeStruct((M, F), jnp.float32),
    )(q, pt_arr, xp, x_skip, W2, b_arr)

    return (out, pos_skip, batch_skip)
